# baseline (speedup 1.0000x reference)
import functools

import jax
import jax.numpy as jnp
from jax.experimental import pallas as pl
from jax.experimental.pallas import tpu as pltpu

N = 512
D = 128
HEADS = 8
C = D // HEADS
LAYERS = 2


def _fused_body(x_ref, w1s_ref, w1d_ref, b1_ref, w2_ref, b2_ref,
                wq_ref, bq_ref, wk_ref, bk_ref, wv_ref, bv_ref,
                we_ref, be_ref, ws_ref, bs_ref, out_ref,
                bscr_ref, zscr_ref):
    x = x_ref[...]

    a = jnp.dot(x, w1s_ref[...], preferred_element_type=jnp.float32,
                precision=jax.lax.Precision.DEFAULT)
    bscr_ref[...] = jnp.dot(x, w1d_ref[...], preferred_element_type=jnp.float32,
                            precision=jax.lax.Precision.DEFAULT)
    b1v = b1_ref[...]
    w2col = w2_ref[...]

    JB = 8

    def zblock(t, carry):
        jb = t * JB
        bblk = bscr_ref[pl.ds(jb, JB), :]
        h = jnp.maximum(a[None, :, :] + bblk[:, None, :] + b1v[None, :, :], 0.0)
        zb = jnp.dot(h.reshape(JB * N, D), w2col,
                     preferred_element_type=jnp.float32,
                     precision=jax.lax.Precision.DEFAULT)
        zscr_ref[pl.ds(jb, JB), :] = zb.reshape(JB, N)
        return carry

    jax.lax.fori_loop(0, N // JB, zblock, 0)
    z = zscr_ref[...] + b2_ref[...]
    ewt = jax.nn.sigmoid(z)
    maskneg = jnp.where(ewt > 0.5, 0.0, -1e30)

    inv_sqrt_c = 1.0 / (C ** 0.5)

    for l in range(LAYERS):
        q = jnp.dot(x, wq_ref[l], preferred_element_type=jnp.float32, precision=jax.lax.Precision.DEFAULT) + bq_ref[l]
        k = jnp.dot(x, wk_ref[l], preferred_element_type=jnp.float32, precision=jax.lax.Precision.DEFAULT) + bk_ref[l]
        v = jnp.dot(x, wv_ref[l], preferred_element_type=jnp.float32, precision=jax.lax.Precision.DEFAULT) + bv_ref[l]
        wef = we_ref[l]
        bef = be_ref[l]

        outs = []
        for h in range(HEADS):
            sl = slice(h * C, (h + 1) * C)
            kh, vh = k[:, sl], v[:, sl]
            qh = q[:, sl] * inv_sqrt_c
            weh = wef[:, sl]
            beh = bef[:, sl]

            s = jax.lax.dot_general(qh, kh, (((1,), (1,)), ((), ())),
                                    preferred_element_type=jnp.float32, precision=jax.lax.Precision.HIGHEST)
            qwe = jax.lax.dot_general(qh, weh, (((1,), (1,)), ((), ())),
                                      preferred_element_type=jnp.float32, precision=jax.lax.Precision.HIGHEST)
            qbe = jax.lax.dot_general(qh, beh, (((1,), (1,)), ((), ())),
                                      preferred_element_type=jnp.float32, precision=jax.lax.Precision.HIGHEST)

            lm = (s + qbe) + ewt * qwe + maskneg
            m = jnp.max(lm, axis=1, keepdims=True)
            m = jnp.where(m < -1e29, 0.0, m)
            ex = jnp.exp(lm - m)
            den = jnp.sum(ex, axis=1, keepdims=True)
            r = 1.0 / (den + 1e-16)
            outv = jnp.dot(ex, vh, preferred_element_type=jnp.float32, precision=jax.lax.Precision.HIGHEST) * r
            sew = jnp.sum(ex * ewt, axis=1, keepdims=True) * r
            sa = den * r
            outs.append(outv + sew * weh + sa * beh)

        attn = jnp.concatenate(outs, axis=1)
        skip = jnp.dot(x, ws_ref[l], preferred_element_type=jnp.float32,
                       precision=jax.lax.Precision.DEFAULT)
        x = x + ((attn + skip) + bs_ref[l])

    out_ref[...] = x


@functools.partial(jax.jit, static_argnames=())
def kernel(mention_features, mention_locations, entity_embeddings,
           W1, b1, W2, b2, Wq, bq, Wk, bk, Wv, bv, We, be, Wskip, bskip):
    del mention_features, mention_locations
    x = entity_embeddings
    w1s = W1[:D]
    w1d = W1[D:]
    b1r = b1.reshape(1, D)
    w2r = W2
    b2r = b2.reshape(1, 1)
    wer = We.reshape(LAYERS, 1, D)
    ber = be.reshape(LAYERS, 1, D)
    bqr = bq.reshape(LAYERS, 1, D)
    bkr = bk.reshape(LAYERS, 1, D)
    bvr = bv.reshape(LAYERS, 1, D)
    bsr = bskip.reshape(LAYERS, 1, D)

    return pl.pallas_call(
        _fused_body,
        out_shape=jax.ShapeDtypeStruct((N, D), jnp.float32),
        scratch_shapes=[pltpu.VMEM((N, D), jnp.float32),
                        pltpu.VMEM((N, N), jnp.float32)],
    )(x, w1s, w1d, b1r, w2r, b2r, Wq, bqr, Wk, bkr, Wv, bvr, wer, ber, Wskip, bsr)


# speedup vs baseline: 1.1231x; 1.1231x over previous
import functools

import jax
import jax.numpy as jnp
from jax.experimental import pallas as pl
from jax.experimental.pallas import tpu as pltpu

N = 512
D = 128
HEADS = 8
C = D // HEADS
LAYERS = 2


def _fused_body(x_ref, w1s_ref, w1d_ref, b1_ref, w2_ref, b2_ref,
                wq_ref, bq_ref, wk_ref, bk_ref, wv_ref, bv_ref,
                we_ref, be_ref, ws_ref, bs_ref, out_ref,
                bscr_ref, zscr_ref):
    x = x_ref[...]

    a = jnp.dot(x, w1s_ref[...], preferred_element_type=jnp.float32,
                precision=jax.lax.Precision.DEFAULT)
    bscr_ref[...] = jnp.dot(x, w1d_ref[...], preferred_element_type=jnp.float32,
                            precision=jax.lax.Precision.DEFAULT)
    b1v = b1_ref[...]
    w2col = w2_ref[...]

    JB = 32

    def zblock(t, carry):
        jb = t * JB
        bblk = bscr_ref[pl.ds(jb, JB), :]
        h = jnp.maximum(a[None, :, :] + bblk[:, None, :] + b1v[None, :, :], 0.0)
        zb = jnp.dot(h.reshape(JB * N, D), w2col,
                     preferred_element_type=jnp.float32,
                     precision=jax.lax.Precision.DEFAULT)
        zscr_ref[pl.ds(jb, JB), :] = zb.reshape(JB, N)
        return carry

    jax.lax.fori_loop(0, N // JB, zblock, 0)
    z = zscr_ref[...] + b2_ref[...]
    ewt = jax.nn.sigmoid(z)
    maskneg = jnp.where(ewt > 0.5, 0.0, -1e30)

    inv_sqrt_c = 1.0 / (C ** 0.5)

    for l in range(LAYERS):
        q = jnp.dot(x, wq_ref[l], preferred_element_type=jnp.float32, precision=jax.lax.Precision.DEFAULT) + bq_ref[l]
        k = jnp.dot(x, wk_ref[l], preferred_element_type=jnp.float32, precision=jax.lax.Precision.DEFAULT) + bk_ref[l]
        v = jnp.dot(x, wv_ref[l], preferred_element_type=jnp.float32, precision=jax.lax.Precision.DEFAULT) + bv_ref[l]
        wef = we_ref[l]
        bef = be_ref[l]

        outs = []
        for h in range(HEADS):
            sl = slice(h * C, (h + 1) * C)
            kh, vh = k[:, sl], v[:, sl]
            qh = q[:, sl] * inv_sqrt_c
            weh = wef[:, sl]
            beh = bef[:, sl]

            s = jax.lax.dot_general(qh, kh, (((1,), (1,)), ((), ())),
                                    preferred_element_type=jnp.float32, precision=jax.lax.Precision.HIGHEST)
            qwe = jax.lax.dot_general(qh, weh, (((1,), (1,)), ((), ())),
                                      preferred_element_type=jnp.float32, precision=jax.lax.Precision.HIGHEST)
            qbe = jax.lax.dot_general(qh, beh, (((1,), (1,)), ((), ())),
                                      preferred_element_type=jnp.float32, precision=jax.lax.Precision.HIGHEST)

            lm = (s + qbe) + ewt * qwe + maskneg
            m = jnp.max(lm, axis=1, keepdims=True)
            m = jnp.where(m < -1e29, 0.0, m)
            ex = jnp.exp(lm - m)
            den = jnp.sum(ex, axis=1, keepdims=True)
            r = 1.0 / (den + 1e-16)
            outv = jnp.dot(ex, vh, preferred_element_type=jnp.float32, precision=jax.lax.Precision.HIGHEST) * r
            sew = jnp.sum(ex * ewt, axis=1, keepdims=True) * r
            sa = den * r
            outs.append(outv + sew * weh + sa * beh)

        attn = jnp.concatenate(outs, axis=1)
        skip = jnp.dot(x, ws_ref[l], preferred_element_type=jnp.float32,
                       precision=jax.lax.Precision.DEFAULT)
        x = x + ((attn + skip) + bs_ref[l])

    out_ref[...] = x


@functools.partial(jax.jit, static_argnames=())
def kernel(mention_features, mention_locations, entity_embeddings,
           W1, b1, W2, b2, Wq, bq, Wk, bk, Wv, bv, We, be, Wskip, bskip):
    del mention_features, mention_locations
    x = entity_embeddings
    w1s = W1[:D]
    w1d = W1[D:]
    b1r = b1.reshape(1, D)
    w2r = W2
    b2r = b2.reshape(1, 1)
    wer = We.reshape(LAYERS, 1, D)
    ber = be.reshape(LAYERS, 1, D)
    bqr = bq.reshape(LAYERS, 1, D)
    bkr = bk.reshape(LAYERS, 1, D)
    bvr = bv.reshape(LAYERS, 1, D)
    bsr = bskip.reshape(LAYERS, 1, D)

    return pl.pallas_call(
        _fused_body,
        out_shape=jax.ShapeDtypeStruct((N, D), jnp.float32),
        scratch_shapes=[pltpu.VMEM((N, D), jnp.float32),
                        pltpu.VMEM((N, N), jnp.float32)],
    )(x, w1s, w1d, b1r, w2r, b2r, Wq, bqr, Wk, bkr, Wv, bvr, wer, ber, Wskip, bsr)


# speedup vs baseline: 1.2569x; 1.1191x over previous
import functools

import jax
import jax.numpy as jnp
from jax.experimental import pallas as pl
from jax.experimental.pallas import tpu as pltpu

N = 512
D = 128
HEADS = 8
C = D // HEADS
LAYERS = 2


def _fused_body(x_ref, w1s_ref, w1d_ref, b1_ref, w2_ref, b2_ref,
                wq_ref, bq_ref, wk_ref, bk_ref, wv_ref, bv_ref,
                we_ref, be_ref, ws_ref, bs_ref, out_ref,
                bscr_ref, zscr_ref):
    x = x_ref[...]

    a = jnp.dot(x, w1s_ref[...], preferred_element_type=jnp.float32,
                precision=jax.lax.Precision.DEFAULT)
    bscr_ref[...] = jnp.dot(x, w1d_ref[...], preferred_element_type=jnp.float32,
                            precision=jax.lax.Precision.DEFAULT)
    b1v = b1_ref[...]
    w2col = w2_ref[...]

    JB = 32

    def zblock(t, carry):
        jb = t * JB
        bblk = bscr_ref[pl.ds(jb, JB), :]
        h = jnp.maximum(a[None, :, :] + bblk[:, None, :] + b1v[None, :, :], 0.0)
        zb = jnp.dot(h.reshape(JB * N, D), w2col,
                     preferred_element_type=jnp.float32,
                     precision=jax.lax.Precision.DEFAULT)
        zscr_ref[pl.ds(jb, JB), :] = zb.reshape(JB, N)
        return carry

    jax.lax.fori_loop(0, N // JB, zblock, 0)
    z = zscr_ref[...] + b2_ref[...]
    ewt = jax.nn.sigmoid(z)
    maskneg = jnp.where(ewt > 0.5, 0.0, -1e30)

    inv_sqrt_c = 1.0 / (C ** 0.5)

    for l in range(LAYERS):
        q = jnp.dot(x, wq_ref[l], preferred_element_type=jnp.float32, precision=jax.lax.Precision.DEFAULT) + bq_ref[l]
        k = jnp.dot(x, wk_ref[l], preferred_element_type=jnp.float32, precision=jax.lax.Precision.DEFAULT) + bk_ref[l]
        v = jnp.dot(x, wv_ref[l], preferred_element_type=jnp.float32, precision=jax.lax.Precision.DEFAULT) + bv_ref[l]
        wef = we_ref[l]
        bef = be_ref[l]

        outs = []
        for h in range(HEADS):
            sl = slice(h * C, (h + 1) * C)
            kh, vh = k[:, sl], v[:, sl]
            qh = q[:, sl] * inv_sqrt_c
            weh = wef[:, sl]
            beh = bef[:, sl]

            s = jax.lax.dot_general(qh, kh, (((1,), (1,)), ((), ())),
                                    preferred_element_type=jnp.float32, precision=jax.lax.Precision.HIGHEST)
            qwe = jax.lax.dot_general(qh, weh, (((1,), (1,)), ((), ())),
                                      preferred_element_type=jnp.float32, precision=jax.lax.Precision.HIGHEST)
            qbe = jax.lax.dot_general(qh, beh, (((1,), (1,)), ((), ())),
                                      preferred_element_type=jnp.float32, precision=jax.lax.Precision.HIGHEST)

            lm = (s + qbe) + ewt * qwe + maskneg
            m = jnp.max(lm, axis=1, keepdims=True)
            m = jnp.where(m < -1e29, 0.0, m)
            ex = jnp.exp(lm - m)
            den = jnp.sum(ex, axis=1, keepdims=True)
            r = 1.0 / (den + 1e-16)
            outv = jnp.dot(ex, vh, preferred_element_type=jnp.float32, precision=jax.lax.Precision.DEFAULT) * r
            sew = jnp.sum(ex * ewt, axis=1, keepdims=True) * r
            sa = den * r
            outs.append(outv + sew * weh + sa * beh)

        attn = jnp.concatenate(outs, axis=1)
        skip = jnp.dot(x, ws_ref[l], preferred_element_type=jnp.float32,
                       precision=jax.lax.Precision.DEFAULT)
        x = x + ((attn + skip) + bs_ref[l])

    out_ref[...] = x


@functools.partial(jax.jit, static_argnames=())
def kernel(mention_features, mention_locations, entity_embeddings,
           W1, b1, W2, b2, Wq, bq, Wk, bk, Wv, bv, We, be, Wskip, bskip):
    del mention_features, mention_locations
    x = entity_embeddings
    w1s = W1[:D]
    w1d = W1[D:]
    b1r = b1.reshape(1, D)
    w2r = W2
    b2r = b2.reshape(1, 1)
    wer = We.reshape(LAYERS, 1, D)
    ber = be.reshape(LAYERS, 1, D)
    bqr = bq.reshape(LAYERS, 1, D)
    bkr = bk.reshape(LAYERS, 1, D)
    bvr = bv.reshape(LAYERS, 1, D)
    bsr = bskip.reshape(LAYERS, 1, D)

    return pl.pallas_call(
        _fused_body,
        out_shape=jax.ShapeDtypeStruct((N, D), jnp.float32),
        scratch_shapes=[pltpu.VMEM((N, D), jnp.float32),
                        pltpu.VMEM((N, N), jnp.float32)],
    )(x, w1s, w1d, b1r, w2r, b2r, Wq, bqr, Wk, bkr, Wv, bvr, wer, ber, Wskip, bsr)


# speedup vs baseline: 1.5641x; 1.2445x over previous
import functools

import jax
import jax.numpy as jnp
from jax.experimental import pallas as pl
from jax.experimental.pallas import tpu as pltpu

N = 512
D = 128
HEADS = 8
C = D // HEADS
LAYERS = 2


def _fused_body(x_ref, w1s_ref, w1d_ref, b1_ref, w2_ref, b2_ref,
                wq_ref, bq_ref, wk_ref, bk_ref, wv_ref, bv_ref,
                we_ref, be_ref, ws_ref, bs_ref, out_ref,
                bscr_ref, zscr_ref):
    x = x_ref[...]

    a = jnp.dot(x, w1s_ref[...], preferred_element_type=jnp.float32,
                precision=jax.lax.Precision.DEFAULT)
    bscr_ref[...] = jnp.dot(x, w1d_ref[...], preferred_element_type=jnp.float32,
                            precision=jax.lax.Precision.DEFAULT)
    b1v = b1_ref[...]
    w2col = w2_ref[...]

    JB = 32

    def zblock(t, carry):
        jb = t * JB
        bblk = bscr_ref[pl.ds(jb, JB), :]
        h = jnp.maximum(a[None, :, :] + bblk[:, None, :] + b1v[None, :, :], 0.0)
        zb = jnp.dot(h.reshape(JB * N, D), w2col,
                     preferred_element_type=jnp.float32,
                     precision=jax.lax.Precision.DEFAULT)
        zscr_ref[pl.ds(jb, JB), :] = zb.reshape(JB, N)
        return carry

    jax.lax.fori_loop(0, N // JB, zblock, 0)
    z = zscr_ref[...] + b2_ref[...]
    ewt = jax.nn.sigmoid(z)
    maskneg = jnp.where(ewt > 0.5, 0.0, -1e30)

    inv_sqrt_c = 1.0 / (C ** 0.5)

    for l in range(LAYERS):
        q = jnp.dot(x, wq_ref[l], preferred_element_type=jnp.float32, precision=jax.lax.Precision.DEFAULT) + bq_ref[l]
        k = jnp.dot(x, wk_ref[l], preferred_element_type=jnp.float32, precision=jax.lax.Precision.DEFAULT) + bk_ref[l]
        v = jnp.dot(x, wv_ref[l], preferred_element_type=jnp.float32, precision=jax.lax.Precision.DEFAULT) + bv_ref[l]
        wef = we_ref[l]
        bef = be_ref[l]

        outs = []
        for h in range(HEADS):
            sl = slice(h * C, (h + 1) * C)
            kh, vh = k[:, sl], v[:, sl]
            qh = q[:, sl] * inv_sqrt_c
            weh = wef[:, sl]
            beh = bef[:, sl]

            s = jax.lax.dot_general(qh, kh, (((1,), (1,)), ((), ())),
                                    preferred_element_type=jnp.float32, precision=jax.lax.Precision.DEFAULT)
            qwe = jax.lax.dot_general(qh, weh, (((1,), (1,)), ((), ())),
                                      preferred_element_type=jnp.float32, precision=jax.lax.Precision.HIGHEST)
            qbe = jax.lax.dot_general(qh, beh, (((1,), (1,)), ((), ())),
                                      preferred_element_type=jnp.float32, precision=jax.lax.Precision.HIGHEST)

            lm = (s + qbe) + ewt * qwe + maskneg
            m = jnp.max(lm, axis=1, keepdims=True)
            m = jnp.where(m < -1e29, 0.0, m)
            ex = jnp.exp(lm - m)
            den = jnp.sum(ex, axis=1, keepdims=True)
            r = 1.0 / (den + 1e-16)
            outv = jnp.dot(ex, vh, preferred_element_type=jnp.float32, precision=jax.lax.Precision.DEFAULT) * r
            sew = jnp.sum(ex * ewt, axis=1, keepdims=True) * r
            sa = den * r
            outs.append(outv + sew * weh + sa * beh)

        attn = jnp.concatenate(outs, axis=1)
        skip = jnp.dot(x, ws_ref[l], preferred_element_type=jnp.float32,
                       precision=jax.lax.Precision.DEFAULT)
        x = x + ((attn + skip) + bs_ref[l])

    out_ref[...] = x


@functools.partial(jax.jit, static_argnames=())
def kernel(mention_features, mention_locations, entity_embeddings,
           W1, b1, W2, b2, Wq, bq, Wk, bk, Wv, bv, We, be, Wskip, bskip):
    del mention_features, mention_locations
    x = entity_embeddings
    w1s = W1[:D]
    w1d = W1[D:]
    b1r = b1.reshape(1, D)
    w2r = W2
    b2r = b2.reshape(1, 1)
    wer = We.reshape(LAYERS, 1, D)
    ber = be.reshape(LAYERS, 1, D)
    bqr = bq.reshape(LAYERS, 1, D)
    bkr = bk.reshape(LAYERS, 1, D)
    bvr = bv.reshape(LAYERS, 1, D)
    bsr = bskip.reshape(LAYERS, 1, D)

    return pl.pallas_call(
        _fused_body,
        out_shape=jax.ShapeDtypeStruct((N, D), jnp.float32),
        scratch_shapes=[pltpu.VMEM((N, D), jnp.float32),
                        pltpu.VMEM((N, N), jnp.float32)],
    )(x, w1s, w1d, b1r, w2r, b2r, Wq, bqr, Wk, bkr, Wv, bvr, wer, ber, Wskip, bsr)
